# baseline (speedup 1.0000x reference)
import functools

import jax
import jax.numpy as jnp
from jax import lax
from jax.experimental import pallas as pl
from jax.experimental.pallas import tpu as pltpu
from jax.experimental.pallas import tpu_sc as plsc

F32 = jnp.float32

_NC = 2
_NS = 16
_LANE = 16
_NW = _NC * _NS
_HCH = 64


def _a_body(x_ref, new_ref, neb_ref, w1h_ref, b1_ref, h_ref, p_ref):
    h = jnp.maximum(x_ref[...] * new_ref[...] + neb_ref[...], 0.0)
    h_ref[...] = h
    p_ref[...] = (
        jnp.dot(h, w1h_ref[...], preferred_element_type=F32) + b1_ref[...]
    )


def _zero_rows(rows_v, nrows, width):
    def _zrow(i, _):
        for j in range(width // _LANE):
            rows_v[i, pl.ds(j * _LANE, _LANE)] = jnp.zeros((_LANE,), F32)
        return 0

    lax.fori_loop(0, nrows, _zrow, 0, unroll=False)


def _zero_my_slice(zbuf, nz, acc, r0, rows_pt):
    off = 0
    while off < rows_pt:
        nn = min(nz, rows_pt - off)
        pltpu.sync_copy(zbuf.at[pl.ds(0, nn)], acc.at[pl.ds(r0 + off, nn)])
        off += nn


def _sc_body(n_acc, nhc,
             p_hbm, rec_hbm, recf_hbm, w1e_hbm,
             s_out,
             rows_v, sdrow, recvf, srcx, dstx, w1ev,
             gsem0, gsem1, gsem2, gsem3,
             ssem0, ssem1, ssem2, ssem3,
             isem0, isem1, isem2, isem3, acc):
    c = lax.axis_index("c")
    s = lax.axis_index("s")
    wid = c * _NS + s
    rows_pt = n_acc // _NS
    gsem = (gsem0, gsem1, gsem2, gsem3)
    ssem = (ssem0, ssem1, ssem2, ssem3)
    isem = (isem0, isem1, isem2, isem3)

    pltpu.sync_copy(w1e_hbm, w1ev)

    _zero_rows(rows_v.at[0], _HCH, rows_v.shape[2])
    r0 = s * rows_pt
    _zero_my_slice(rows_v.at[0], _HCH, acc, r0, rows_pt)
    plsc.subcore_barrier()

    def _issue_idx(t, b):
        pltpu.async_copy(rec_hbm.at[wid, t, 0], sdrow.at[b], isem[b])
        pltpu.async_copy(recf_hbm.at[wid, t], recvf.at[b], isem[b])

    def _wait_idx(b):
        pltpu.make_async_copy(rec_hbm.at[wid, 0, 0], sdrow.at[b],
                              isem[b]).wait()
        pltpu.make_async_copy(recf_hbm.at[wid, 0], recvf.at[b],
                              isem[b]).wait()

    def _unpack_idx(b):
        for k in range(4):
            sl = pl.ds(k * _LANE, _LANE)
            srcx[b, sl] = sdrow[b, sl]
            dstx[b, sl] = sdrow[b, pl.ds(_HCH + k * _LANE, _LANE)]

    def _issue_gather(b):
        pltpu.async_copy(p_hbm.at[srcx.at[b]], rows_v.at[b], gsem[b])

    def _wait_gather(b):
        pltpu.make_async_copy(
            p_hbm.at[srcx.at[b]], rows_v.at[b], gsem[b]
        ).wait()

    def _issue_scatter(b):
        pltpu.async_copy(rows_v.at[b], acc.at[dstx.at[b]], ssem[b],
                         add=True)

    def _wait_scatter(b):
        pltpu.make_async_copy(
            rows_v.at[b], acc.at[dstx.at[b]], ssem[b]
        ).wait()

    def _compute(b):
        @plsc.parallel_loop(0, _HCH // _LANE, unroll=2)
        def _grp16(gg):
            o = pl.multiple_of(gg * _LANE, _LANE)
            o2 = pl.multiple_of(_HCH + gg * _LANE, _LANE)
            av = (recvf[b, 0, pl.ds(o, _LANE)], recvf[b, 0, pl.ds(o2, _LANE)],
                  recvf[b, 1, pl.ds(o, _LANE)], recvf[b, 1, pl.ds(o2, _LANE)])
            for k in range(_LANE):
                i = gg * _LANE + k
                eb = [lax.broadcast(av[t][k], (_LANE,)) for t in range(4)]
                for j in range(8):
                    sl = pl.ds(j * _LANE, _LANE)
                    v = rows_v[b, i, sl]
                    m01 = eb[0] * w[0][j] + eb[1] * w[1][j]
                    m23 = eb[2] * w[2][j] + eb[3] * w[3][j]
                    rows_v[b, i, sl] = jnp.maximum(v + (m01 + m23), 0.0)

    _issue_idx(0, 0)
    _issue_idx(1, 1)
    _issue_idx(2, 2)
    _wait_idx(0)
    _unpack_idx(0)
    _wait_idx(1)
    _unpack_idx(1)
    w = [[w1ev[ci, pl.ds(j * _LANE, _LANE)] for j in range(8)] for ci in range(4)]
    _issue_gather(0)
    _issue_gather(1)

    def _ring(kk, _):
        for q in range(4):
            b = q
            t = kk * 4 + q
            b2 = (q + 2) % 4

            _wait_gather(b)

            @pl.when(t + 2 < nhc)
            def _():
                _wait_idx(b2)
                _unpack_idx(b2)

            _compute(b)
            _issue_scatter(b)

            @pl.when(t + 2 < nhc)
            def _():
                _issue_gather(b2)

            b3 = (q + 3) % 4

            @pl.when(t + 3 < nhc)
            def _():
                @pl.when(t >= 1)
                def _():
                    _wait_scatter(b3)

                _issue_idx(t + 3, b3)

        return 0

    lax.fori_loop(0, nhc // 4, _ring, 0, unroll=False)

    for b in range(4):
        _wait_scatter(b)

    plsc.subcore_barrier()

    pltpu.sync_copy(acc.at[pl.ds(r0, rows_pt)], s_out.at[c, pl.ds(r0, rows_pt)])


def _sc_edge_pass(p, rec, recf, w1e, n_acc):
    nhc = rec.shape[1]
    h = p.shape[1]
    mesh = plsc.VectorSubcoreMesh(core_axis_name="c", subcore_axis_name="s")
    body = functools.partial(_sc_body, n_acc, nhc)
    sems = [pltpu.SemaphoreType.DMA] * 12
    return pl.kernel(
        body,
        out_type=jax.ShapeDtypeStruct((_NC, n_acc, h), F32),
        mesh=mesh,
        scratch_types=(
            pltpu.VMEM((4, _HCH, h), F32),
            pltpu.VMEM((4, 128), jnp.int32),
            pltpu.VMEM((4, 2, 128), F32),
            pltpu.VMEM((4, _HCH), jnp.int32),
            pltpu.VMEM((4, _HCH), jnp.int32),
            pltpu.VMEM((4, h), F32),
            *sems,
            pltpu.VMEM_SHARED((n_acc, h), F32),
        ),
    )(p, rec, recf, w1e)


def _sc_cnt_body(n_acc, nhc, hdim,
                 rec_hbm, cnt_out,
                 onesv, sdrow, dstx,
                 ssem0, ssem1, ssem2, ssem3,
                 isem0, isem1, isem2, isem3, cacc):
    c = lax.axis_index("c")
    s = lax.axis_index("s")
    wid = c * _NS + s
    rows_pt = n_acc // _NS
    ssem = (ssem0, ssem1, ssem2, ssem3)
    isem = (isem0, isem1, isem2, isem3)

    _zero_rows(onesv, _HCH, hdim)
    r0 = s * rows_pt
    _zero_my_slice(onesv, _HCH, cacc, r0, rows_pt)

    def _ones(i, _):
        onesv[i, pl.ds(0, _LANE)] = jnp.ones((_LANE,), F32)
        return 0

    lax.fori_loop(0, _HCH, _ones, 0, unroll=False)
    plsc.subcore_barrier()

    def _issue_idx(t, b):
        pltpu.async_copy(rec_hbm.at[wid, t, 0], sdrow.at[b], isem[b])

    def _wait_idx(b):
        pltpu.make_async_copy(rec_hbm.at[wid, 0, 0], sdrow.at[b],
                              isem[b]).wait()

    def _unpack_idx(b):
        for k in range(4):
            sl = pl.ds(k * _LANE, _LANE)
            dstx[b, sl] = sdrow[b, pl.ds(_HCH + k * _LANE, _LANE)]

    def _issue_scatter(b):
        pltpu.async_copy(onesv, cacc.at[dstx.at[b]], ssem[b], add=True)

    def _wait_scatter(b):
        pltpu.make_async_copy(onesv, cacc.at[dstx.at[b]], ssem[b]).wait()

    _issue_idx(0, 0)
    _issue_idx(1, 1)
    _wait_idx(0)
    _unpack_idx(0)

    def _ring(kk, _):
        for q in range(4):
            b = q
            t = kk * 4 + q
            bn = (q + 1) % 4

            @pl.when(t + 1 < nhc)
            def _():
                _wait_idx(bn)
                _unpack_idx(bn)

            _issue_scatter(b)

            b2 = (q + 2) % 4

            @pl.when(t + 2 < nhc)
            def _():
                @pl.when(t >= 2)
                def _():
                    _wait_scatter(b2)

                _issue_idx(t + 2, b2)

        return 0

    lax.fori_loop(0, nhc // 4, _ring, 0, unroll=False)

    for b in range(4):
        _wait_scatter(b)

    plsc.subcore_barrier()
    pltpu.sync_copy(cacc.at[pl.ds(r0, rows_pt)], cnt_out.at[c, pl.ds(r0, rows_pt)])


def _sc_count_pass(rec, n_acc, hdim):
    nhc = rec.shape[1]
    mesh = plsc.VectorSubcoreMesh(core_axis_name="c", subcore_axis_name="s")
    body = functools.partial(_sc_cnt_body, n_acc, nhc, hdim)
    sems = [pltpu.SemaphoreType.DMA] * 8
    return pl.kernel(
        body,
        out_type=jax.ShapeDtypeStruct((_NC, n_acc, hdim), F32),
        mesh=mesh,
        scratch_types=(
            pltpu.VMEM((_HCH, hdim), F32),
            pltpu.VMEM((4, 128), jnp.int32),
            pltpu.VMEM((4, _HCH), jnp.int32),
            *sems,
            pltpu.VMEM_SHARED((n_acc, hdim), F32),
        ),
    )(rec)


def kernel(x, edge_index, edge_attr, ne_W, ne_b,
           l0_em_W1, l0_em_b1, l0_em_W2, l0_em_b2,
           l0_um_W1, l0_um_b1, l0_um_W2, l0_um_b2,
           l1_em_W1, l1_em_b1, l1_em_W2, l1_em_b2,
           l1_um_W1, l1_um_b1, l1_um_W2, l1_um_b2,
           out_W1, out_b1, out_W2, out_b2, vel_W1, vel_b1, vel_W2, vel_b2):
    n = x.shape[0]
    e = edge_index.shape[1]
    hdim = ne_W.shape[1]

    nhc = -(-e // (_NW * _HCH * 4)) * 4
    e_pad = nhc * _NW * _HCH
    pad = e_pad - e
    n_acc = -(-(n + _LANE) // (_NS * 8)) * (_NS * 8)
    pad_ids = jnp.arange(pad, dtype=jnp.int32) % _LANE
    src = jnp.concatenate([edge_index[0], pad_ids]).reshape(_NW, nhc, 1, _HCH)
    dst = (jnp.concatenate([edge_index[1], n + pad_ids])
           .reshape(_NW, nhc, 1, _HCH))
    rec = jnp.concatenate([src, dst], axis=3)
    atp = jnp.concatenate([edge_attr, jnp.zeros((pad, 4), F32)], axis=0)
    att = atp.T.reshape(4, _NW, nhc, _HCH)
    recf = att.transpose(1, 2, 0, 3).reshape(_NW, nhc, 2, 128)

    l0w1h, l0w1e = l0_em_W1[:hdim], l0_em_W1[hdim:]
    l1w1h, l1w1e = l1_em_W1[:hdim], l1_em_W1[hdim:]
    l0umh, l0umm = l0_um_W1[:hdim], l0_um_W1[hdim:]
    l1umh, l1umm = l1_um_W1[:hdim], l1_um_W1[hdim:]
    wcat = jnp.concatenate([out_W1, vel_W1], axis=1)
    bcat = jnp.concatenate([out_b1, vel_b1])[None, :]
    w2big = jnp.zeros((2 * hdim, hdim), F32)
    w2big = w2big.at[:hdim, :3].set(out_W2).at[hdim:, 3:5].set(vel_W2)
    bbig = jnp.zeros((hdim,), F32).at[:3].set(out_b2).at[3:5].set(vel_b2)[None, :]

    bn = 2000
    grid = n // bn
    row_spec = pl.BlockSpec((bn, hdim), lambda i: (i, 0))
    col1_spec = pl.BlockSpec((bn, 1), lambda i: (i, 0))
    s_spec = pl.BlockSpec((_NC, bn, hdim), lambda i: (0, i, 0))
    c_spec = s_spec

    def wspec(a):
        return pl.BlockSpec(a.shape, lambda i: (0,) * a.ndim)

    neb = ne_b[None, :]
    l0b1 = l0_em_b1[None, :]
    h0, p0 = pl.pallas_call(
        _a_body,
        grid=(grid,),
        in_specs=[col1_spec, wspec(ne_W), wspec(neb), wspec(l0w1h), wspec(l0b1)],
        out_specs=[row_spec, row_spec],
        out_shape=[
            jax.ShapeDtypeStruct((n, hdim), F32),
            jax.ShapeDtypeStruct((n, hdim), F32),
        ],
    )(x, ne_W, neb, l0w1h, l0b1)

    cnt = _sc_count_pass(rec, n_acc, hdim)

    s0 = _sc_edge_pass(p0, rec, recf, l0w1e, n_acc)

    l0emb2 = l0_em_b2[None, :]
    l0umb1 = l0_um_b1[None, :]
    l0umb2 = l0_um_b2[None, :]
    l1b1 = l1_em_b1[None, :]

    def _c0_body(s_ref, c_ref, h_ref, emw2_ref, emb2_ref, umw1h_ref, umw1m_ref,
                 umb1_ref, umw2_ref, umb2_ref, nxtw_ref, nxtb_ref,
                 hn_ref, p_ref):
        S = s_ref[0] + s_ref[1]
        cntc = c_ref[0, :, 0:1] + c_ref[1, :, 0:1]
        mm = (
            jnp.dot(S / jnp.maximum(cntc, 1.0), emw2_ref[...],
                    preferred_element_type=F32)
            + emb2_ref[...]
        )
        mean = jnp.where(cntc > 0.0, mm, 0.0)
        h = h_ref[...]
        t = jnp.maximum(
            jnp.dot(h, umw1h_ref[...], preferred_element_type=F32)
            + jnp.dot(mean, umw1m_ref[...], preferred_element_type=F32)
            + umb1_ref[...],
            0.0,
        )
        hn = h + jnp.dot(t, umw2_ref[...], preferred_element_type=F32) + umb2_ref[...]
        hn_ref[...] = hn
        p_ref[...] = jnp.dot(hn, nxtw_ref[...], preferred_element_type=F32) + nxtb_ref[...]

    h1, p1 = pl.pallas_call(
        _c0_body,
        grid=(grid,),
        in_specs=[s_spec, c_spec, row_spec, wspec(l0_em_W2), wspec(l0emb2),
                  wspec(l0umh), wspec(l0umm), wspec(l0umb1), wspec(l0_um_W2),
                  wspec(l0umb2), wspec(l1w1h), wspec(l1b1)],
        out_specs=[row_spec, row_spec],
        out_shape=[
            jax.ShapeDtypeStruct((n, hdim), F32),
            jax.ShapeDtypeStruct((n, hdim), F32),
        ],
    )(s0, cnt, h0, l0_em_W2, l0emb2, l0umh, l0umm, l0umb1, l0_um_W2, l0umb2,
      l1w1h, l1b1)

    s1 = _sc_edge_pass(p1, rec, recf, l1w1e, n_acc)

    l1emb2 = l1_em_b2[None, :]
    l1umb1 = l1_um_b1[None, :]
    l1umb2 = l1_um_b2[None, :]

    def _c1_body(s_ref, c_ref, h_ref, emw2_ref, emb2_ref, umw1h_ref, umw1m_ref,
                 umb1_ref, umw2_ref, umb2_ref, wcat_ref, bcat_ref, w2big_ref,
                 bbig_ref, big_ref):
        S = s_ref[0] + s_ref[1]
        cntc = c_ref[0, :, 0:1] + c_ref[1, :, 0:1]
        mm = (
            jnp.dot(S / jnp.maximum(cntc, 1.0), emw2_ref[...],
                    preferred_element_type=F32)
            + emb2_ref[...]
        )
        mean = jnp.where(cntc > 0.0, mm, 0.0)
        h = h_ref[...]
        t = jnp.maximum(
            jnp.dot(h, umw1h_ref[...], preferred_element_type=F32)
            + jnp.dot(mean, umw1m_ref[...], preferred_element_type=F32)
            + umb1_ref[...],
            0.0,
        )
        hn = h + jnp.dot(t, umw2_ref[...], preferred_element_type=F32) + umb2_ref[...]
        a = jnp.maximum(
            jnp.dot(hn, wcat_ref[...], preferred_element_type=F32) + bcat_ref[...],
            0.0,
        )
        big_ref[...] = jnp.dot(a, w2big_ref[...], preferred_element_type=F32) + bbig_ref[...]

    big = pl.pallas_call(
        _c1_body,
        grid=(grid,),
        in_specs=[s_spec, c_spec, row_spec, wspec(l1_em_W2), wspec(l1emb2),
                  wspec(l1umh), wspec(l1umm), wspec(l1umb1), wspec(l1_um_W2),
                  wspec(l1umb2), wspec(wcat), wspec(bcat), wspec(w2big),
                  wspec(bbig)],
        out_specs=[row_spec],
        out_shape=[jax.ShapeDtypeStruct((n, hdim), F32)],
    )(s1, cnt, h1, l1_em_W2, l1emb2, l1umh, l1umm, l1umb1, l1_um_W2, l1umb2,
      wcat, bcat, w2big, bbig)[0]

    return (big[:, :3], big[:, 3:5])


# speedup vs baseline: 1.2858x; 1.2858x over previous
import functools

import jax
import jax.numpy as jnp
from jax import lax
from jax.experimental import pallas as pl
from jax.experimental.pallas import tpu as pltpu
from jax.experimental.pallas import tpu_sc as plsc

F32 = jnp.float32

_NC = 2
_NS = 16
_LANE = 16
_NW = _NC * _NS
_HCH = 64


def _a_body(x_ref, new_ref, neb_ref, w1h_ref, b1_ref, h_ref, p_ref):
    h = jnp.maximum(x_ref[...] * new_ref[...] + neb_ref[...], 0.0)
    h_ref[...] = h
    p_ref[...] = (
        jnp.dot(h, w1h_ref[...], preferred_element_type=F32) + b1_ref[...]
    )


def _zero_rows(rows_v, nrows, width):
    def _zrow(i, _):
        for j in range(width // _LANE):
            rows_v[i, pl.ds(j * _LANE, _LANE)] = jnp.zeros((_LANE,), F32)
        return 0

    lax.fori_loop(0, nrows, _zrow, 0, unroll=False)


def _zero_my_slice(zbuf, nz, acc, r0, rows_pt):
    off = 0
    while off < rows_pt:
        nn = min(nz, rows_pt - off)
        pltpu.sync_copy(zbuf.at[pl.ds(0, nn)], acc.at[pl.ds(r0 + off, nn)])
        off += nn


def _sc_body(n_acc, nhc,
             p_hbm, rec_hbm, recf_hbm, w1e_hbm,
             s_out,
             rows_v, sdrow, recvf, srcx, dstx, w1ev,
             gsem0, gsem1, gsem2, gsem3,
             ssem0, ssem1, ssem2, ssem3,
             isem0, isem1, isem2, isem3, acc):
    c = lax.axis_index("c")
    s = lax.axis_index("s")
    wid = c * _NS + s
    rows_pt = n_acc // _NS
    gsem = (gsem0, gsem1, gsem2, gsem3)
    ssem = (ssem0, ssem1, ssem2, ssem3)
    isem = (isem0, isem1, isem2, isem3)

    pltpu.sync_copy(w1e_hbm, w1ev)

    _zero_rows(rows_v.at[0], _HCH, rows_v.shape[2])
    r0 = s * rows_pt
    _zero_my_slice(rows_v.at[0], _HCH, acc, r0, rows_pt)
    plsc.subcore_barrier()

    def _issue_idx(t, b):
        pltpu.async_copy(rec_hbm.at[wid, t, 0], sdrow.at[b], isem[b])
        pltpu.async_copy(recf_hbm.at[wid, t], recvf.at[b], isem[b])

    def _wait_idx(b):
        pltpu.make_async_copy(rec_hbm.at[wid, 0, 0], sdrow.at[b],
                              isem[b]).wait()
        pltpu.make_async_copy(recf_hbm.at[wid, 0], recvf.at[b],
                              isem[b]).wait()

    def _unpack_idx(b):
        for k in range(4):
            sl = pl.ds(k * _LANE, _LANE)
            srcx[b, sl] = sdrow[b, sl]
            dstx[b, sl] = sdrow[b, pl.ds(_HCH + k * _LANE, _LANE)]

    def _issue_gather(b):
        pltpu.async_copy(p_hbm.at[srcx.at[b]], rows_v.at[b], gsem[b])

    def _wait_gather(b):
        pltpu.make_async_copy(
            p_hbm.at[srcx.at[b]], rows_v.at[b], gsem[b]
        ).wait()

    def _issue_scatter(b):
        pltpu.async_copy(rows_v.at[b], acc.at[dstx.at[b]], ssem[b],
                         add=True)

    def _wait_scatter(b):
        pltpu.make_async_copy(
            rows_v.at[b], acc.at[dstx.at[b]], ssem[b]
        ).wait()

    def _compute(b):
        def _grp16(gg, _):
            o = pl.multiple_of(gg * _LANE, _LANE)
            o2 = pl.multiple_of(_HCH + gg * _LANE, _LANE)
            av = (recvf[b, 0, pl.ds(o, _LANE)], recvf[b, 0, pl.ds(o2, _LANE)],
                  recvf[b, 1, pl.ds(o, _LANE)], recvf[b, 1, pl.ds(o2, _LANE)])
            for k in range(_LANE):
                i = gg * _LANE + k
                e0, e1, e2, e3 = (av[t][k] for t in range(4))
                for j in range(8):
                    sl = pl.ds(j * _LANE, _LANE)
                    v = rows_v[b, i, sl]
                    v = (v + e0 * w[0][j] + e1 * w[1][j]
                         + e2 * w[2][j] + e3 * w[3][j])
                    rows_v[b, i, sl] = jnp.maximum(v, 0.0)
            return 0

        lax.fori_loop(0, _HCH // _LANE, _grp16, 0, unroll=False)

    _issue_idx(0, 0)
    _issue_idx(1, 1)
    _issue_idx(2, 2)
    _wait_idx(0)
    _unpack_idx(0)
    _wait_idx(1)
    _unpack_idx(1)
    w = [[w1ev[ci, pl.ds(j * _LANE, _LANE)] for j in range(8)] for ci in range(4)]
    _issue_gather(0)
    _issue_gather(1)

    def _ring(kk, _):
        for q in range(4):
            b = q
            t = kk * 4 + q
            b2 = (q + 2) % 4

            _wait_gather(b)

            @pl.when(t + 2 < nhc)
            def _():
                _wait_idx(b2)
                _unpack_idx(b2)

            _compute(b)
            _issue_scatter(b)

            @pl.when(t + 2 < nhc)
            def _():
                _issue_gather(b2)

            b3 = (q + 3) % 4

            @pl.when(t + 3 < nhc)
            def _():
                @pl.when(t >= 1)
                def _():
                    _wait_scatter(b3)

                _issue_idx(t + 3, b3)

        return 0

    lax.fori_loop(0, nhc // 4, _ring, 0, unroll=False)

    for b in range(4):
        _wait_scatter(b)

    plsc.subcore_barrier()

    pltpu.sync_copy(acc.at[pl.ds(r0, rows_pt)], s_out.at[c, pl.ds(r0, rows_pt)])


def _sc_edge_pass(p, rec, recf, w1e, n_acc):
    nhc = rec.shape[1]
    h = p.shape[1]
    mesh = plsc.VectorSubcoreMesh(core_axis_name="c", subcore_axis_name="s")
    body = functools.partial(_sc_body, n_acc, nhc)
    sems = [pltpu.SemaphoreType.DMA] * 12
    return pl.kernel(
        body,
        out_type=jax.ShapeDtypeStruct((_NC, n_acc, h), F32),
        mesh=mesh,
        scratch_types=(
            pltpu.VMEM((4, _HCH, h), F32),
            pltpu.VMEM((4, 128), jnp.int32),
            pltpu.VMEM((4, 2, 128), F32),
            pltpu.VMEM((4, _HCH), jnp.int32),
            pltpu.VMEM((4, _HCH), jnp.int32),
            pltpu.VMEM((4, h), F32),
            *sems,
            pltpu.VMEM_SHARED((n_acc, h), F32),
        ),
    )(p, rec, recf, w1e)


def _sc_cnt_body(n_acc, nhc, hdim,
                 rec_hbm, cnt_out,
                 onesv, sdrow, dstx,
                 ssem0, ssem1, ssem2, ssem3,
                 isem0, isem1, isem2, isem3, cacc):
    c = lax.axis_index("c")
    s = lax.axis_index("s")
    wid = c * _NS + s
    rows_pt = n_acc // _NS
    ssem = (ssem0, ssem1, ssem2, ssem3)
    isem = (isem0, isem1, isem2, isem3)

    _zero_rows(onesv, _HCH, hdim)
    r0 = s * rows_pt
    _zero_my_slice(onesv, _HCH, cacc, r0, rows_pt)

    def _ones(i, _):
        onesv[i, pl.ds(0, _LANE)] = jnp.ones((_LANE,), F32)
        return 0

    lax.fori_loop(0, _HCH, _ones, 0, unroll=False)
    plsc.subcore_barrier()

    def _issue_idx(t, b):
        pltpu.async_copy(rec_hbm.at[wid, t, 0], sdrow.at[b], isem[b])

    def _wait_idx(b):
        pltpu.make_async_copy(rec_hbm.at[wid, 0, 0], sdrow.at[b],
                              isem[b]).wait()

    def _unpack_idx(b):
        for k in range(4):
            sl = pl.ds(k * _LANE, _LANE)
            dstx[b, sl] = sdrow[b, pl.ds(_HCH + k * _LANE, _LANE)]

    def _issue_scatter(b):
        pltpu.async_copy(onesv, cacc.at[dstx.at[b]], ssem[b], add=True)

    def _wait_scatter(b):
        pltpu.make_async_copy(onesv, cacc.at[dstx.at[b]], ssem[b]).wait()

    _issue_idx(0, 0)
    _issue_idx(1, 1)
    _wait_idx(0)
    _unpack_idx(0)

    def _ring(kk, _):
        for q in range(4):
            b = q
            t = kk * 4 + q
            bn = (q + 1) % 4

            @pl.when(t + 1 < nhc)
            def _():
                _wait_idx(bn)
                _unpack_idx(bn)

            _issue_scatter(b)

            b2 = (q + 2) % 4

            @pl.when(t + 2 < nhc)
            def _():
                @pl.when(t >= 2)
                def _():
                    _wait_scatter(b2)

                _issue_idx(t + 2, b2)

        return 0

    lax.fori_loop(0, nhc // 4, _ring, 0, unroll=False)

    for b in range(4):
        _wait_scatter(b)

    plsc.subcore_barrier()
    pltpu.sync_copy(cacc.at[pl.ds(r0, rows_pt)], cnt_out.at[c, pl.ds(r0, rows_pt)])


def _sc_count_pass(rec, n_acc, hdim):
    nhc = rec.shape[1]
    mesh = plsc.VectorSubcoreMesh(core_axis_name="c", subcore_axis_name="s")
    body = functools.partial(_sc_cnt_body, n_acc, nhc, hdim)
    sems = [pltpu.SemaphoreType.DMA] * 8
    return pl.kernel(
        body,
        out_type=jax.ShapeDtypeStruct((_NC, n_acc, hdim), F32),
        mesh=mesh,
        scratch_types=(
            pltpu.VMEM((_HCH, hdim), F32),
            pltpu.VMEM((4, 128), jnp.int32),
            pltpu.VMEM((4, _HCH), jnp.int32),
            *sems,
            pltpu.VMEM_SHARED((n_acc, hdim), F32),
        ),
    )(rec)


def kernel(x, edge_index, edge_attr, ne_W, ne_b,
           l0_em_W1, l0_em_b1, l0_em_W2, l0_em_b2,
           l0_um_W1, l0_um_b1, l0_um_W2, l0_um_b2,
           l1_em_W1, l1_em_b1, l1_em_W2, l1_em_b2,
           l1_um_W1, l1_um_b1, l1_um_W2, l1_um_b2,
           out_W1, out_b1, out_W2, out_b2, vel_W1, vel_b1, vel_W2, vel_b2):
    n = x.shape[0]
    e = edge_index.shape[1]
    hdim = ne_W.shape[1]

    nhc = -(-e // (_NW * _HCH * 4)) * 4
    e_pad = nhc * _NW * _HCH
    pad = e_pad - e
    n_acc = -(-(n + _LANE) // (_NS * 8)) * (_NS * 8)
    pad_ids = jnp.arange(pad, dtype=jnp.int32) % _LANE
    src = jnp.concatenate([edge_index[0], pad_ids]).reshape(_NW, nhc, 1, _HCH)
    dst = (jnp.concatenate([edge_index[1], n + pad_ids])
           .reshape(_NW, nhc, 1, _HCH))
    rec = jnp.concatenate([src, dst], axis=3)
    atp = jnp.concatenate([edge_attr, jnp.zeros((pad, 4), F32)], axis=0)
    att = atp.T.reshape(4, _NW, nhc, _HCH)
    recf = att.transpose(1, 2, 0, 3).reshape(_NW, nhc, 2, 128)

    l0w1h, l0w1e = l0_em_W1[:hdim], l0_em_W1[hdim:]
    l1w1h, l1w1e = l1_em_W1[:hdim], l1_em_W1[hdim:]
    l0umh, l0umm = l0_um_W1[:hdim], l0_um_W1[hdim:]
    l1umh, l1umm = l1_um_W1[:hdim], l1_um_W1[hdim:]
    wcat = jnp.concatenate([out_W1, vel_W1], axis=1)
    bcat = jnp.concatenate([out_b1, vel_b1])[None, :]
    w2big = jnp.zeros((2 * hdim, hdim), F32)
    w2big = w2big.at[:hdim, :3].set(out_W2).at[hdim:, 3:5].set(vel_W2)
    bbig = jnp.zeros((hdim,), F32).at[:3].set(out_b2).at[3:5].set(vel_b2)[None, :]

    bn = 2000
    grid = n // bn
    row_spec = pl.BlockSpec((bn, hdim), lambda i: (i, 0))
    col1_spec = pl.BlockSpec((bn, 1), lambda i: (i, 0))
    s_spec = pl.BlockSpec((_NC, bn, hdim), lambda i: (0, i, 0))
    c_spec = s_spec

    def wspec(a):
        return pl.BlockSpec(a.shape, lambda i: (0,) * a.ndim)

    neb = ne_b[None, :]
    l0b1 = l0_em_b1[None, :]
    h0, p0 = pl.pallas_call(
        _a_body,
        grid=(grid,),
        in_specs=[col1_spec, wspec(ne_W), wspec(neb), wspec(l0w1h), wspec(l0b1)],
        out_specs=[row_spec, row_spec],
        out_shape=[
            jax.ShapeDtypeStruct((n, hdim), F32),
            jax.ShapeDtypeStruct((n, hdim), F32),
        ],
    )(x, ne_W, neb, l0w1h, l0b1)

    cnt = _sc_count_pass(rec, n_acc, hdim)

    s0 = _sc_edge_pass(p0, rec, recf, l0w1e, n_acc)

    l0emb2 = l0_em_b2[None, :]
    l0umb1 = l0_um_b1[None, :]
    l0umb2 = l0_um_b2[None, :]
    l1b1 = l1_em_b1[None, :]

    def _c0_body(s_ref, c_ref, h_ref, emw2_ref, emb2_ref, umw1h_ref, umw1m_ref,
                 umb1_ref, umw2_ref, umb2_ref, nxtw_ref, nxtb_ref,
                 hn_ref, p_ref):
        S = s_ref[0] + s_ref[1]
        cntc = c_ref[0, :, 0:1] + c_ref[1, :, 0:1]
        mm = (
            jnp.dot(S / jnp.maximum(cntc, 1.0), emw2_ref[...],
                    preferred_element_type=F32)
            + emb2_ref[...]
        )
        mean = jnp.where(cntc > 0.0, mm, 0.0)
        h = h_ref[...]
        t = jnp.maximum(
            jnp.dot(h, umw1h_ref[...], preferred_element_type=F32)
            + jnp.dot(mean, umw1m_ref[...], preferred_element_type=F32)
            + umb1_ref[...],
            0.0,
        )
        hn = h + jnp.dot(t, umw2_ref[...], preferred_element_type=F32) + umb2_ref[...]
        hn_ref[...] = hn
        p_ref[...] = jnp.dot(hn, nxtw_ref[...], preferred_element_type=F32) + nxtb_ref[...]

    h1, p1 = pl.pallas_call(
        _c0_body,
        grid=(grid,),
        in_specs=[s_spec, c_spec, row_spec, wspec(l0_em_W2), wspec(l0emb2),
                  wspec(l0umh), wspec(l0umm), wspec(l0umb1), wspec(l0_um_W2),
                  wspec(l0umb2), wspec(l1w1h), wspec(l1b1)],
        out_specs=[row_spec, row_spec],
        out_shape=[
            jax.ShapeDtypeStruct((n, hdim), F32),
            jax.ShapeDtypeStruct((n, hdim), F32),
        ],
    )(s0, cnt, h0, l0_em_W2, l0emb2, l0umh, l0umm, l0umb1, l0_um_W2, l0umb2,
      l1w1h, l1b1)

    s1 = _sc_edge_pass(p1, rec, recf, l1w1e, n_acc)

    l1emb2 = l1_em_b2[None, :]
    l1umb1 = l1_um_b1[None, :]
    l1umb2 = l1_um_b2[None, :]

    def _c1_body(s_ref, c_ref, h_ref, emw2_ref, emb2_ref, umw1h_ref, umw1m_ref,
                 umb1_ref, umw2_ref, umb2_ref, wcat_ref, bcat_ref, w2big_ref,
                 bbig_ref, big_ref):
        S = s_ref[0] + s_ref[1]
        cntc = c_ref[0, :, 0:1] + c_ref[1, :, 0:1]
        mm = (
            jnp.dot(S / jnp.maximum(cntc, 1.0), emw2_ref[...],
                    preferred_element_type=F32)
            + emb2_ref[...]
        )
        mean = jnp.where(cntc > 0.0, mm, 0.0)
        h = h_ref[...]
        t = jnp.maximum(
            jnp.dot(h, umw1h_ref[...], preferred_element_type=F32)
            + jnp.dot(mean, umw1m_ref[...], preferred_element_type=F32)
            + umb1_ref[...],
            0.0,
        )
        hn = h + jnp.dot(t, umw2_ref[...], preferred_element_type=F32) + umb2_ref[...]
        a = jnp.maximum(
            jnp.dot(hn, wcat_ref[...], preferred_element_type=F32) + bcat_ref[...],
            0.0,
        )
        big_ref[...] = jnp.dot(a, w2big_ref[...], preferred_element_type=F32) + bbig_ref[...]

    big = pl.pallas_call(
        _c1_body,
        grid=(grid,),
        in_specs=[s_spec, c_spec, row_spec, wspec(l1_em_W2), wspec(l1emb2),
                  wspec(l1umh), wspec(l1umm), wspec(l1umb1), wspec(l1_um_W2),
                  wspec(l1umb2), wspec(wcat), wspec(bcat), wspec(w2big),
                  wspec(bbig)],
        out_specs=[row_spec],
        out_shape=[jax.ShapeDtypeStruct((n, hdim), F32)],
    )(s1, cnt, h1, l1_em_W2, l1emb2, l1umh, l1umm, l1umb1, l1_um_W2, l1umb2,
      wcat, bcat, w2big, bbig)[0]

    return (big[:, :3], big[:, 3:5])
